# Initial kernel scaffold; baseline (speedup 1.0000x reference)
import jax
import jax.numpy as jnp
from jax.experimental import pallas as pl


def kernel(x, table):
    raise NotImplementedError("write your pallas kernel here")


# speedup vs baseline: 4.9489x; 4.9489x over previous
import functools

import jax
import jax.numpy as jnp
from jax import lax
from jax.experimental import pallas as pl
from jax.experimental.pallas import tpu as pltpu
from jax.experimental.pallas import tpu_sc as plsc

_B = 16384
_L = 200
_DIM = 32
_ROWS = _B * _L
_GRP = 128
_NGROUPS = _ROWS // _GRP

_NC = 2
_NS = 16
_NW = _NC * _NS
_GPW = _NGROUPS // _NW

_K = 16
_ITERS = _GPW // _K


def _gather_body(idx_hbm, table_hbm, out_hbm, idx_v, rows_v, sem):
    wid = lax.axis_index("s") * _NC + lax.axis_index("c")
    base = wid * _GPW

    def slab(i, carry):
        g0 = base + i * _K
        pltpu.sync_copy(idx_hbm.at[pl.ds(g0, _K)], idx_v)
        copies = [
            pltpu.async_copy(table_hbm.at[idx_v.at[j]], rows_v.at[j], sem)
            for j in range(_K)
        ]
        for c in copies:
            c.wait()
        pltpu.sync_copy(rows_v, out_hbm.at[pl.ds(g0, _K)])
        return carry

    lax.fori_loop(0, _ITERS, slab, 0)


@jax.jit
def _embedding_lookup(idx, table):
    mesh = plsc.VectorSubcoreMesh(core_axis_name="c", subcore_axis_name="s")
    return pl.kernel(
        _gather_body,
        mesh=mesh,
        out_type=jax.ShapeDtypeStruct((_NGROUPS, _GRP, _DIM), jnp.float32),
        scratch_types=[
            pltpu.VMEM((_K, _GRP), jnp.int32),
            pltpu.VMEM((_K, _GRP, _DIM), jnp.float32),
            pltpu.SemaphoreType.DMA,
        ],
        compiler_params=pltpu.CompilerParams(use_tc_tiling_on_sc=False),
    )(idx, table)


def kernel(x, table):
    idx = x.reshape(_NGROUPS, _GRP).astype(jnp.int32)
    out = _embedding_lookup(idx, table)
    return out.reshape(_B, _L, _DIM)


# speedup vs baseline: 4.9693x; 1.0041x over previous
import functools

import jax
import jax.numpy as jnp
from jax import lax
from jax.experimental import pallas as pl
from jax.experimental.pallas import tpu as pltpu
from jax.experimental.pallas import tpu_sc as plsc

_B = 16384
_L = 200
_DIM = 32
_ROWS = _B * _L
_GRP = 128
_NGROUPS = _ROWS // _GRP

_NC = 2
_NS = 16
_NW = _NC * _NS
_GPW = _NGROUPS // _NW

_K = 10
_NSLAB = _GPW // _K


def _gather_body(idx_hbm, table_hbm, out_hbm,
                 idx0, idx1, rows0, rows1, sem0, sem1):
    wid = lax.axis_index("s") * _NC + lax.axis_index("c")
    base = wid * _GPW
    idx_v = (idx0, idx1)
    rows_v = (rows0, rows1)
    sems = (sem0, sem1)

    def fire(b, s):
        g0 = base + s * _K
        pltpu.sync_copy(idx_hbm.at[pl.ds(g0, _K)], idx_v[b])
        for j in range(_K):
            pltpu.async_copy(table_hbm.at[idx_v[b].at[j]],
                             rows_v[b].at[j], sems[b])

    def drain_and_write(b, s):
        for j in range(_K):
            pltpu.make_async_copy(table_hbm.at[idx_v[b].at[j]],
                                  rows_v[b].at[j], sems[b]).wait()
        pltpu.sync_copy(rows_v[b], out_hbm.at[pl.ds(base + s * _K, _K)])

    fire(0, 0)

    def step(i, carry):
        s0 = 2 * i
        fire(1, s0 + 1)
        drain_and_write(0, s0)
        fire(0, s0 + 2)
        drain_and_write(1, s0 + 1)
        return carry

    lax.fori_loop(0, _NSLAB // 2 - 1, step, 0)
    fire(1, _NSLAB - 1)
    drain_and_write(0, _NSLAB - 2)
    drain_and_write(1, _NSLAB - 1)


@jax.jit
def _embedding_lookup(idx, table):
    mesh = plsc.VectorSubcoreMesh(core_axis_name="c", subcore_axis_name="s")
    return pl.kernel(
        _gather_body,
        mesh=mesh,
        out_type=jax.ShapeDtypeStruct((_NGROUPS, _GRP, _DIM), jnp.float32),
        scratch_types=[
            pltpu.VMEM((_K, _GRP), jnp.int32),
            pltpu.VMEM((_K, _GRP), jnp.int32),
            pltpu.VMEM((_K, _GRP, _DIM), jnp.float32),
            pltpu.VMEM((_K, _GRP, _DIM), jnp.float32),
            pltpu.SemaphoreType.DMA,
            pltpu.SemaphoreType.DMA,
        ],
        compiler_params=pltpu.CompilerParams(use_tc_tiling_on_sc=False),
    )(idx, table)


def kernel(x, table):
    idx = x.reshape(_NGROUPS, _GRP).astype(jnp.int32)
    out = _embedding_lookup(idx, table)
    return out.reshape(_B, _L, _DIM)


# speedup vs baseline: 4.9846x; 1.0031x over previous
import functools

import jax
import jax.numpy as jnp
from jax import lax
from jax.experimental import pallas as pl
from jax.experimental.pallas import tpu as pltpu
from jax.experimental.pallas import tpu_sc as plsc

_B = 16384
_L = 200
_DIM = 32

_NC = 2
_NS = 16
_NW = _NC * _NS
_BPW = _B // _NW

_SB = 8
_NSLAB = _BPW // _SB
_SPLITS = ((0, 128), (128, 72))


def _gather_body(x_hbm, table_hbm, out_hbm,
                 idx0, idx1, rows0, rows1, sem0, sem1):
    wid = lax.axis_index("s") * _NC + lax.axis_index("c")
    base = wid * _BPW
    idx_v = (idx0, idx1)
    rows_v = (rows0, rows1)
    sems = (sem0, sem1)

    def fire(b, s):
        b0 = base + s * _SB
        pltpu.sync_copy(x_hbm.at[pl.ds(b0, _SB)], idx_v[b])
        for r in range(_SB):
            for off, n in _SPLITS:
                pltpu.async_copy(
                    table_hbm.at[idx_v[b].at[r, pl.ds(off, n)]],
                    rows_v[b].at[r, pl.ds(off, n)], sems[b])

    def drain_and_write(b, s):
        for r in range(_SB):
            for off, n in _SPLITS:
                pltpu.make_async_copy(
                    table_hbm.at[idx_v[b].at[r, pl.ds(off, n)]],
                    rows_v[b].at[r, pl.ds(off, n)], sems[b]).wait()
        pltpu.sync_copy(rows_v[b], out_hbm.at[pl.ds(base + s * _SB, _SB)])

    fire(0, 0)

    def step(i, carry):
        s0 = 2 * i
        fire(1, s0 + 1)
        drain_and_write(0, s0)
        fire(0, s0 + 2)
        drain_and_write(1, s0 + 1)
        return carry

    lax.fori_loop(0, _NSLAB // 2 - 1, step, 0)
    fire(1, _NSLAB - 1)
    drain_and_write(0, _NSLAB - 2)
    drain_and_write(1, _NSLAB - 1)


@jax.jit
def _embedding_lookup(idx, table):
    mesh = plsc.VectorSubcoreMesh(core_axis_name="c", subcore_axis_name="s")
    return pl.kernel(
        _gather_body,
        mesh=mesh,
        out_type=jax.ShapeDtypeStruct((_B, _L, _DIM), jnp.float32),
        scratch_types=[
            pltpu.VMEM((_SB, _L), jnp.int32),
            pltpu.VMEM((_SB, _L), jnp.int32),
            pltpu.VMEM((_SB, _L, _DIM), jnp.float32),
            pltpu.VMEM((_SB, _L, _DIM), jnp.float32),
            pltpu.SemaphoreType.DMA,
            pltpu.SemaphoreType.DMA,
        ],
        compiler_params=pltpu.CompilerParams(use_tc_tiling_on_sc=False),
    )(idx, table)


def kernel(x, table):
    return _embedding_lookup(x.astype(jnp.int32), table)
